# Initial kernel scaffold; baseline (speedup 1.0000x reference)
import jax
import jax.numpy as jnp
from jax.experimental import pallas as pl


def kernel(x, g):
    raise NotImplementedError("write your pallas kernel here")


# speedup vs baseline: 1.5746x; 1.5746x over previous
import functools

import jax
import jax.numpy as jnp
from jax import lax
from jax.experimental import pallas as pl
from jax.experimental.pallas import tpu as pltpu
from jax.experimental.pallas import tpu_sc as plsc

N, C, H, W = 4, 192, 224, 224
HO, WO = 224, 224
P = N * HO * WO
HW = H * W
L = 16
NC, NS = 2, 16
NW = NC * NS
PPT = P // NW
CH = 64
NCHUNK = PPT // CH
CCH = C // L


def _lane_bcast(v, j):
    idx = jnp.full((L,), j, jnp.int32)
    return lax.gather(
        v, idx[:, None],
        dimension_numbers=lax.GatherDimensionNumbers(
            offset_dims=(), collapsed_slice_dims=(0,), start_index_map=(0,)),
        slice_sizes=(1,), mode=lax.GatherScatterMode.PROMISE_IN_BOUNDS)


def _make_sc_kernel():
    mesh = plsc.VectorSubcoreMesh(core_axis_name="c", subcore_axis_name="s")

    @functools.partial(
        pl.kernel,
        mesh=mesh,
        compiler_params=pltpu.CompilerParams(use_tc_tiling_on_sc=False),
        out_type=jax.ShapeDtypeStruct((P, C), jnp.float32),
        scratch_types=[
            pltpu.VMEM((CH,), jnp.float32),
            pltpu.VMEM((CH,), jnp.float32),
            pltpu.VMEM((CH,), jnp.int32),
            pltpu.VMEM((CH,), jnp.int32),
            pltpu.VMEM((CH,), jnp.int32),
            pltpu.VMEM((CH,), jnp.int32),
            pltpu.VMEM((CH,), jnp.float32),
            pltpu.VMEM((CH,), jnp.float32),
            pltpu.VMEM((CH,), jnp.float32),
            pltpu.VMEM((CH,), jnp.float32),
            pltpu.VMEM((CH, C), jnp.float32),
            pltpu.VMEM((CH, C), jnp.float32),
            pltpu.VMEM((CH, C), jnp.float32),
            pltpu.VMEM((CH, C), jnp.float32),
            pltpu.VMEM((CH, C), jnp.float32),
            pltpu.SemaphoreType.DMA,
        ],
    )
    def grid_sample_sc(xt_hbm, gx_hbm, gy_hbm, out_hbm,
                       gx_v, gy_v, i00, i01, i10, i11,
                       w00, w01, w10, w11,
                       r00, r01, r10, r11, out_v, sem):
        wid = lax.axis_index("s") * NC + lax.axis_index("c")
        tbase = wid * PPT

        def chunk_body(ci, carry):
            base = tbase + ci * CH
            pltpu.sync_copy(gx_hbm.at[pl.ds(base, CH)], gx_v)
            pltpu.sync_copy(gy_hbm.at[pl.ds(base, CH)], gy_v)

            for gidx in range(CH // L):
                gx = gx_v[pl.ds(gidx * L, L)]
                gy = gy_v[pl.ds(gidx * L, L)]
                ix = (gx + 1.0) * ((W - 1) / 2.0)
                iy = (gy + 1.0) * ((H - 1) / 2.0)
                ix0 = ix.astype(jnp.int32)
                ix0f = ix0.astype(jnp.float32)
                negx = ix0f > ix
                ix0 = jnp.where(negx, ix0 - 1, ix0)
                ix0f = jnp.where(negx, ix0f - 1.0, ix0f)
                iy0 = iy.astype(jnp.int32)
                iy0f = iy0.astype(jnp.float32)
                negy = iy0f > iy
                iy0 = jnp.where(negy, iy0 - 1, iy0)
                iy0f = jnp.where(negy, iy0f - 1.0, iy0f)
                fx = ix - ix0f
                fy = iy - iy0f
                wx0 = 1.0 - fx
                wy0 = 1.0 - fy
                ix1 = ix0 + 1
                iy1 = iy0 + 1
                mx0 = jnp.where(ix0 >= 0, 1.0, 0.0) * jnp.where(ix0 <= W - 1, 1.0, 0.0)
                mx1 = jnp.where(ix1 >= 0, 1.0, 0.0) * jnp.where(ix1 <= W - 1, 1.0, 0.0)
                my0 = jnp.where(iy0 >= 0, 1.0, 0.0) * jnp.where(iy0 <= H - 1, 1.0, 0.0)
                my1 = jnp.where(iy1 >= 0, 1.0, 0.0) * jnp.where(iy1 <= H - 1, 1.0, 0.0)
                cx0 = jnp.minimum(jnp.maximum(ix0, 0), W - 1)
                cx1 = jnp.minimum(jnp.maximum(ix1, 0), W - 1)
                cy0 = jnp.minimum(jnp.maximum(iy0, 0), H - 1)
                cy1 = jnp.minimum(jnp.maximum(iy1, 0), H - 1)
                lane = lax.iota(jnp.int32, L)
                pix = base + gidx * L + lane
                nb = lax.div(pix, HO * WO) * HW
                i00[pl.ds(gidx * L, L)] = nb + cy0 * W + cx0
                i01[pl.ds(gidx * L, L)] = nb + cy0 * W + cx1
                i10[pl.ds(gidx * L, L)] = nb + cy1 * W + cx0
                i11[pl.ds(gidx * L, L)] = nb + cy1 * W + cx1
                w00[pl.ds(gidx * L, L)] = wy0 * wx0 * (my0 * mx0)
                w01[pl.ds(gidx * L, L)] = wy0 * fx * (my0 * mx1)
                w10[pl.ds(gidx * L, L)] = fy * wx0 * (my1 * mx0)
                w11[pl.ds(gidx * L, L)] = fy * fx * (my1 * mx1)

            d0 = pltpu.async_copy(xt_hbm.at[i00], r00, sem)
            d1 = pltpu.async_copy(xt_hbm.at[i01], r01, sem)
            d2 = pltpu.async_copy(xt_hbm.at[i10], r10, sem)
            d3 = pltpu.async_copy(xt_hbm.at[i11], r11, sem)
            d0.wait()
            d1.wait()
            d2.wait()
            d3.wait()

            for gidx in range(CH // L):
                wv00 = w00[pl.ds(gidx * L, L)]
                wv01 = w01[pl.ds(gidx * L, L)]
                wv10 = w10[pl.ds(gidx * L, L)]
                wv11 = w11[pl.ds(gidx * L, L)]

                def pix_body(j, carry2, wv00=wv00, wv01=wv01,
                             wv10=wv10, wv11=wv11, gidx=gidx):
                    b00 = _lane_bcast(wv00, j)
                    b01 = _lane_bcast(wv01, j)
                    b10 = _lane_bcast(wv10, j)
                    b11 = _lane_bcast(wv11, j)
                    i = gidx * L + j
                    for cc in range(CCH):
                        s = pl.ds(cc * L, L)
                        acc = (r00[i, s] * b00 + r01[i, s] * b01
                               + r10[i, s] * b10 + r11[i, s] * b11)
                        out_v[i, s] = acc
                    return carry2

                lax.fori_loop(0, L, pix_body, 0)
            pltpu.sync_copy(out_v, out_hbm.at[pl.ds(base, CH)])
            return carry

        lax.fori_loop(0, NCHUNK, chunk_body, 0)

    return grid_sample_sc


_grid_sample_sc = _make_sc_kernel()


def kernel(x, g):
    xt = jnp.transpose(x, (0, 2, 3, 1)).reshape(N * H * W, C)
    gx = g[..., 0].reshape(P)
    gy = g[..., 1].reshape(P)
    out_t = _grid_sample_sc(xt, gx, gy)
    return jnp.transpose(out_t.reshape(N, HO, WO, C), (0, 3, 1, 2))
